# baseline (speedup 1.0000x reference)
import functools

import jax
import jax.numpy as jnp
from jax import lax
from jax.experimental import pallas as pl
from jax.experimental.pallas import tpu as pltpu
from jax.experimental.pallas import tpu_sc as plsc

DIM = 16

_info = plsc.get_sparse_core_info()
_NC, _NS = _info.num_cores, _info.num_subcores
_NW = _NC * _NS

_CHUNK = 2048


def _make_gather(total, dim):
    assert total % (_NW * _CHUNK) == 0
    b_per_w = total // _NW
    n_steps = b_per_w // _CHUNK
    mesh = plsc.VectorSubcoreMesh(core_axis_name="c", subcore_axis_name="s")

    @functools.partial(
        pl.kernel,
        mesh=mesh,
        out_type=jax.ShapeDtypeStruct((total, dim), jnp.float32),
        scratch_types=[
            pltpu.VMEM((_CHUNK,), jnp.int32),
            pltpu.VMEM((_CHUNK, dim), jnp.float32),
            pltpu.SemaphoreType.DMA,
        ],
        compiler_params=pltpu.CompilerParams(use_tc_tiling_on_sc=False),
    )
    def gather_kernel(idx_hbm, table_hbm, out_hbm, idx_v, rows_v, sem):
        wid = lax.axis_index("s") * _NC + lax.axis_index("c")
        base = wid * b_per_w

        def step(i, carry):
            off = base + i * _CHUNK
            pltpu.sync_copy(idx_hbm.at[pl.ds(off, _CHUNK)], idx_v)
            pltpu.async_copy(table_hbm.at[idx_v], rows_v, sem).wait()
            pltpu.sync_copy(rows_v, out_hbm.at[pl.ds(off, _CHUNK)])
            return carry

        lax.fori_loop(0, n_steps, step, 0, unroll=False)

    return gather_kernel


@jax.jit
def kernel(indices, table):
    batch, fields = indices.shape
    total = batch * fields
    flat_idx = indices.reshape(total)
    out = _make_gather(total, DIM)(flat_idx, table)
    return out.reshape(batch, fields, DIM)


# speedup vs baseline: 1.0108x; 1.0108x over previous
import functools

import jax
import jax.numpy as jnp
from jax import lax
from jax.experimental import pallas as pl
from jax.experimental.pallas import tpu as pltpu
from jax.experimental.pallas import tpu_sc as plsc

DIM = 16

_info = plsc.get_sparse_core_info()
_NC, _NS = _info.num_cores, _info.num_subcores
_NW = _NC * _NS

_CHUNK = 1600
_NBUF = 4


def _make_gather(total, dim):
    assert total % (_NW * _CHUNK * _NBUF) == 0
    b_per_w = total // _NW
    n_steps = b_per_w // _CHUNK
    n_outer = n_steps // _NBUF
    mesh = plsc.VectorSubcoreMesh(core_axis_name="c", subcore_axis_name="s")

    @functools.partial(
        pl.kernel,
        mesh=mesh,
        out_type=jax.ShapeDtypeStruct((total, dim), jnp.float32),
        scratch_types=[
            pltpu.VMEM((_NBUF, _CHUNK), jnp.int32),
            pltpu.VMEM((_NBUF, _CHUNK, dim), jnp.float32),
        ]
        + [pltpu.SemaphoreType.DMA] * (2 * _NBUF),
        compiler_params=pltpu.CompilerParams(use_tc_tiling_on_sc=False),
    )
    def gather_kernel(idx_hbm, table_hbm, out_hbm, idx_v, rows_v, *sems):
        gsems = sems[:_NBUF]
        osems = sems[_NBUF:]
        wid = lax.axis_index("s") * _NC + lax.axis_index("c")
        base = wid * b_per_w

        def start_gather(i, b):
            pltpu.sync_copy(idx_hbm.at[pl.ds(base + i * _CHUNK, _CHUNK)],
                            idx_v.at[b])
            pltpu.async_copy(table_hbm.at[idx_v.at[b]], rows_v.at[b], gsems[b])

        for b in range(_NBUF):
            start_gather(b, b)

        def outer(o, carry):
            for b in range(_NBUF):
                i = o * _NBUF + b
                pltpu.make_async_copy(
                    table_hbm.at[idx_v.at[b]], rows_v.at[b], gsems[b]).wait()
                store = pltpu.make_async_copy(
                    rows_v.at[b],
                    out_hbm.at[pl.ds(base + i * _CHUNK, _CHUNK)],
                    osems[b])
                store.start()
                store.wait()

                @pl.when(o + 1 < n_outer)
                def _():
                    start_gather(i + _NBUF, b)
            return carry

        lax.fori_loop(0, n_outer, outer, 0, unroll=False)

    return gather_kernel


@jax.jit
def kernel(indices, table):
    batch, fields = indices.shape
    total = batch * fields
    flat_idx = indices.reshape(total)
    out = _make_gather(total, DIM)(flat_idx, table)
    return out.reshape(batch, fields, DIM)


# speedup vs baseline: 3.4682x; 3.4311x over previous
import functools

import jax
import jax.numpy as jnp
from jax import lax
from jax.experimental import pallas as pl
from jax.experimental.pallas import tpu as pltpu
from jax.experimental.pallas import tpu_sc as plsc

DIM = 16

_info = plsc.get_sparse_core_info()
_NC, _NS = _info.num_cores, _info.num_subcores
_NW = _NC * _NS

_CB = 512
_NBUF = 4


def _make_gather(fields, batch, dim):
    n_chunks = batch // _CB
    n_items = fields * n_chunks
    per_w = n_items // _NW
    n_outer = per_w // _NBUF
    assert n_chunks * _CB == batch and per_w * _NW == n_items
    assert n_outer * _NBUF == per_w
    mesh = plsc.VectorSubcoreMesh(core_axis_name="c", subcore_axis_name="s")

    @functools.partial(
        pl.kernel,
        mesh=mesh,
        out_type=jax.ShapeDtypeStruct((fields * dim, batch), jnp.float32),
        scratch_types=[
            pltpu.VMEM((_NBUF, _CB), jnp.int32),
            pltpu.VMEM((_NBUF, _CB, dim), jnp.float32),
        ]
        + [pltpu.VMEM((dim * _CB,), jnp.float32)] * _NBUF
        + [pltpu.SemaphoreType.DMA] * (2 * _NBUF),
        compiler_params=pltpu.CompilerParams(
            use_tc_tiling_on_sc=False, needs_layout_passes=False),
    )
    def gather_kernel(idx_hbm, table_hbm, out_hbm, idx_v, rows_v, *rest):
        outt = rest[:_NBUF]
        gsems = rest[_NBUF:2 * _NBUF]
        osems = rest[2 * _NBUF:]
        wid = lax.axis_index("s") * _NC + lax.axis_index("c")
        base = wid * per_w
        iota16 = lax.iota(jnp.int32, 16)

        def start_gather(k, b):
            pltpu.sync_copy(idx_hbm.at[pl.ds((base + k) * _CB, _CB)],
                            idx_v.at[b])
            pltpu.async_copy(table_hbm.at[idx_v.at[b]], rows_v.at[b], gsems[b])

        for b in range(_NBUF):
            start_gather(b, b)

        def outer(o, carry):
            for b in range(_NBUF):
                k = o * _NBUF + b
                item = base + k
                f = item // n_chunks
                b0 = (item % n_chunks) * _CB
                pltpu.make_async_copy(
                    table_hbm.at[idx_v.at[b]], rows_v.at[b], gsems[b]).wait()

                def blk(i, c):
                    rows = iota16 + i * 16
                    for j in range(dim):
                        v = plsc.load_gather(
                            rows_v.at[b],
                            [rows, jnp.full((16,), j, jnp.int32)])
                        outt[b][pl.ds(j * _CB + i * 16, 16)] = v
                    return c

                lax.fori_loop(0, _CB // 16, blk, 0, unroll=False)

                stores = [
                    pltpu.make_async_copy(
                        outt[b].at[pl.ds(j * _CB, _CB)],
                        out_hbm.at[f * dim + j, pl.ds(b0, _CB)],
                        osems[b])
                    for j in range(dim)
                ]
                for s in stores:
                    s.start()
                for s in stores:
                    s.wait()

                @pl.when(o + 1 < n_outer)
                def _():
                    start_gather(k + _NBUF, b)
            return carry

        lax.fori_loop(0, n_outer, outer, 0, unroll=False)

    return gather_kernel


@jax.jit
def kernel(indices, table):
    batch, fields = indices.shape
    idx_flat = indices.T.reshape(fields * batch)
    out2 = _make_gather(fields, batch, DIM)(idx_flat, table)
    return jnp.transpose(out2.reshape(fields, DIM, batch), (2, 0, 1))


# speedup vs baseline: 3.8743x; 1.1171x over previous
import functools

import jax
import jax.numpy as jnp
from jax import lax
from jax.experimental import pallas as pl
from jax.experimental.pallas import tpu as pltpu
from jax.experimental.pallas import tpu_sc as plsc

DIM = 16

_info = plsc.get_sparse_core_info()
_NC, _NS = _info.num_cores, _info.num_subcores
_NW = _NC * _NS

_CB = 512
_NBUF = 4


def _make_gather(fields, batch, dim):
    n_chunks = batch // _CB
    n_items = fields * n_chunks
    per_w = n_items // _NW
    n_outer = per_w // _NBUF
    assert n_chunks * _CB == batch and per_w * _NW == n_items
    assert n_outer * _NBUF == per_w
    mesh = plsc.VectorSubcoreMesh(core_axis_name="c", subcore_axis_name="s")

    @functools.partial(
        pl.kernel,
        mesh=mesh,
        out_type=jax.ShapeDtypeStruct(
            (fields * dim // 8, batch // 128, 8, 128), jnp.float32),
        scratch_types=[
            pltpu.VMEM((_NBUF, _CB), jnp.int32),
            pltpu.VMEM((_NBUF, _CB, dim), jnp.float32),
        ]
        + [pltpu.VMEM((dim * _CB // 128, 128), jnp.float32)] * _NBUF
        + [pltpu.SemaphoreType.DMA] * (2 * _NBUF),
        compiler_params=pltpu.CompilerParams(
            use_tc_tiling_on_sc=False, needs_layout_passes=False),
    )
    def gather_kernel(idx_hbm, table_hbm, out_hbm, idx_v, rows_v, *rest):
        outt = rest[:_NBUF]
        gsems = rest[_NBUF:2 * _NBUF]
        osems = rest[2 * _NBUF:]
        wid = lax.axis_index("s") * _NC + lax.axis_index("c")
        base = wid * per_w
        iota16 = lax.iota(jnp.int32, 16)

        def start_gather(k, b):
            pltpu.sync_copy(idx_hbm.at[pl.ds((base + k) * _CB, _CB)],
                            idx_v.at[b])
            pltpu.async_copy(table_hbm.at[idx_v.at[b]], rows_v.at[b], gsems[b])

        for b in range(_NBUF):
            start_gather(b, b)

        def outer(o, carry):
            for b in range(_NBUF):
                k = o * _NBUF + b
                item = base + k
                f = item // n_chunks
                b0 = (item % n_chunks) * _CB
                pltpu.make_async_copy(
                    table_hbm.at[idx_v.at[b]], rows_v.at[b], gsems[b]).wait()

                def blk(i, c):
                    rows = iota16 + i * 16
                    for j in range(dim):
                        v = plsc.load_gather(
                            rows_v.at[b],
                            [rows, jnp.full((16,), j, jnp.int32)])
                        outt[b][j * (_CB // 128) + i // 8,
                                pl.ds((i % 8) * 16, 16)] = v
                    return c

                lax.fori_loop(0, _CB // 16, blk, 0, unroll=False)

                stores = [
                    pltpu.make_async_copy(
                        outt[b].at[pl.ds(j * (_CB // 128), _CB // 128), :],
                        out_hbm.at[f * (dim // 8) + j // 8,
                                   pl.ds(b0 // 128, _CB // 128), j % 8, :],
                        osems[b])
                    for j in range(dim)
                ]
                for s in stores:
                    s.start()
                for s in stores:
                    s.wait()

                @pl.when(o + 1 < n_outer)
                def _():
                    start_gather(k + _NBUF, b)
            return carry

        lax.fori_loop(0, n_outer, outer, 0, unroll=False)

    return gather_kernel


@jax.jit
def kernel(indices, table):
    batch, fields = indices.shape
    idx_flat = indices.T.reshape(fields * batch)
    out4 = _make_gather(fields, batch, DIM)(idx_flat, table)
    out5 = out4.reshape(fields, DIM // 8, batch // 128, 8, 128)
    return jnp.transpose(out5, (2, 4, 0, 1, 3)).reshape(batch, fields, DIM)


# speedup vs baseline: 3.9854x; 1.0287x over previous
import functools

import jax
import jax.numpy as jnp
from jax import lax
from jax.experimental import pallas as pl
from jax.experimental.pallas import tpu as pltpu
from jax.experimental.pallas import tpu_sc as plsc

DIM = 16

_info = plsc.get_sparse_core_info()
_NC, _NS = _info.num_cores, _info.num_subcores
_NW = _NC * _NS

_CB = 512
_NBUF = 4


def _make_gather(fields, batch, dim):
    n_chunks = batch // _CB
    n_items = fields * n_chunks
    per_w = n_items // _NW
    n_outer = per_w // _NBUF
    assert n_chunks * _CB == batch and per_w * _NW == n_items
    assert n_outer * _NBUF == per_w
    mesh = plsc.VectorSubcoreMesh(core_axis_name="c", subcore_axis_name="s")

    @functools.partial(
        pl.kernel,
        mesh=mesh,
        out_type=jax.ShapeDtypeStruct(
            (fields * dim // 8, batch // 128, 8, 128), jnp.float32),
        scratch_types=[
            pltpu.VMEM((_NBUF, _CB), jnp.int32),
            pltpu.VMEM((_NBUF, _CB, dim), jnp.float32),
        ]
        + [pltpu.VMEM((dim * _CB // 128, 128), jnp.float32)] * _NBUF
        + [pltpu.SemaphoreType.DMA] * (2 * _NBUF),
        compiler_params=pltpu.CompilerParams(
            use_tc_tiling_on_sc=False, needs_layout_passes=False),
    )
    def gather_kernel(idx_hbm, table_hbm, out_hbm, idx_v, rows_v, *rest):
        outt = rest[:_NBUF]
        gsems = rest[_NBUF:2 * _NBUF]
        osems = rest[2 * _NBUF:]
        wid = lax.axis_index("s") * _NC + lax.axis_index("c")
        base = wid * per_w
        iota16 = lax.iota(jnp.int32, 16)

        def start_gather(k, b):
            pltpu.sync_copy(idx_hbm.at[pl.ds((base + k) * _CB, _CB)],
                            idx_v.at[b])
            pltpu.async_copy(table_hbm.at[idx_v.at[b]], rows_v.at[b], gsems[b])

        for b in range(_NBUF):
            start_gather(b, b)

        def outer(o, carry):
            for b in range(_NBUF):
                k = o * _NBUF + b
                item = base + k
                f = item // n_chunks
                b0 = (item % n_chunks) * _CB
                pltpu.make_async_copy(
                    table_hbm.at[idx_v.at[b]], rows_v.at[b], gsems[b]).wait()

                def blk(i, c):
                    rows = iota16 + i * 16
                    for j in range(dim):
                        v = plsc.load_gather(
                            rows_v.at[b],
                            [rows, jnp.full((16,), j, jnp.int32)])
                        outt[b][j * (_CB // 128) + i // 8,
                                pl.ds((i % 8) * 16, 16)] = v
                    return c

                lax.fori_loop(0, _CB // 16, blk, 0, unroll=False)

                stores = [
                    pltpu.make_async_copy(
                        outt[b].at[pl.ds(j * (_CB // 128), _CB // 128), :],
                        out_hbm.at[f * (dim // 8) + j // 8,
                                   pl.ds(b0 // 128, _CB // 128), j % 8, :],
                        osems[b])
                    for j in range(dim)
                ]
                for s in stores:
                    s.start()
                for s in stores:
                    s.wait()

                @pl.when(o + 1 < n_outer)
                def _():
                    start_gather(k + _NBUF, b)
            return carry

        lax.fori_loop(0, n_outer, outer, 0, unroll=False)

    return gather_kernel


_RC = 128
_RBUF = 4


def _make_retile(vocab, dim):
    n_full = vocab // _RC
    tail = vocab - n_full * _RC
    max_t = (n_full + _NW - 1) // _NW
    mesh = plsc.VectorSubcoreMesh(core_axis_name="c", subcore_axis_name="s")

    @functools.partial(
        pl.kernel,
        mesh=mesh,
        out_type=jax.ShapeDtypeStruct((vocab * dim // 128, 128),
                                      jnp.float32),
        scratch_types=[pltpu.VMEM((dim, _RC), jnp.float32)] * (2 * _RBUF)
        + [pltpu.SemaphoreType.DMA] * (2 * _RBUF)
        + [pltpu.VMEM((dim, 64), jnp.float32),
           pltpu.VMEM((8, 128), jnp.float32)],
        compiler_params=pltpu.CompilerParams(
            use_tc_tiling_on_sc=True, needs_layout_passes=False),
    )
    def retile_kernel(tt_hbm, out_hbm, *rest):
        ins = rest[:_RBUF]
        outt = rest[_RBUF:2 * _RBUF]
        gsems = rest[2 * _RBUF:3 * _RBUF]
        osems = rest[3 * _RBUF:4 * _RBUF]
        tin, tout = rest[4 * _RBUF], rest[4 * _RBUF + 1]
        wid = lax.axis_index("s") * _NC + lax.axis_index("c")
        iota16 = lax.iota(jnp.int32, dim)

        def chunk_id(t):
            return wid + t * _NW

        def start_load(t, b):
            c = chunk_id(t)

            @pl.when(c < n_full)
            def _():
                pltpu.make_async_copy(
                    tt_hbm.at[:, pl.ds(c * _RC, _RC)], ins[b],
                    gsems[b]).start()

        for b in range(min(_RBUF, max_t)):
            start_load(b, b)

        def transpose(src, dst, width):
            def body(v, carry):
                vals = plsc.load_gather(
                    src, [iota16, jnp.full((dim,), v, jnp.int32)])
                dst[v // 8, pl.ds((v % 8) * dim, dim)] = vals
                return carry

            lax.fori_loop(0, width, body, 0, unroll=False)

        n_rounds = (max_t + _RBUF - 1) // _RBUF

        def outer(o, carry):
            for b in range(_RBUF):
                t = o * _RBUF + b
                c = chunk_id(t)

                @pl.when(c < n_full)
                def _():
                    pltpu.make_async_copy(
                        tt_hbm.at[:, pl.ds(c * _RC, _RC)], ins[b],
                        gsems[b]).wait()
                    transpose(ins[b], outt[b], _RC)
                    st = pltpu.make_async_copy(
                        outt[b],
                        out_hbm.at[pl.ds(c * (_RC * dim // 128),
                                         _RC * dim // 128), :],
                        osems[b])
                    st.start()
                    st.wait()

                @pl.when(t + _RBUF < max_t)
                def _():
                    start_load(t + _RBUF, b)
            return carry

        lax.fori_loop(0, n_rounds, outer, 0, unroll=False)

        if tail:
            @pl.when(wid == 0)
            def _():
                pltpu.sync_copy(tt_hbm.at[:, pl.ds(n_full * _RC, tail)], tin)
                transpose(tin, tout, tail)
                pltpu.sync_copy(
                    tout,
                    out_hbm.at[pl.ds(n_full * (_RC * dim // 128),
                                     tail * dim // 128), :])

    return retile_kernel


@jax.jit
def kernel(indices, table):
    batch, fields = indices.shape
    vocab = table.shape[0]
    idx_flat = indices.T.reshape(fields * batch)
    table_rm = _make_retile(vocab, DIM)(table.T).reshape(vocab, DIM)
    out4 = _make_gather(fields, batch, DIM)(idx_flat, table_rm)
    out5 = out4.reshape(fields, DIM // 8, batch // 128, 8, 128)
    return jnp.transpose(out5, (2, 4, 0, 1, 3)).reshape(batch, fields, DIM)


# speedup vs baseline: 4.1353x; 1.0376x over previous
import functools

import jax
import jax.numpy as jnp
from jax import lax
from jax.experimental import pallas as pl
from jax.experimental.pallas import tpu as pltpu
from jax.experimental.pallas import tpu_sc as plsc

DIM = 16

_info = plsc.get_sparse_core_info()
_NC, _NS = _info.num_cores, _info.num_subcores
_NW = _NC * _NS

_CB = 512
_NBUF = 4


def _make_gather(fields, batch, dim):
    n_chunks = batch // _CB
    n_items = fields * n_chunks
    per_w = n_items // _NW
    n_outer = per_w // _NBUF
    assert n_chunks * _CB == batch and per_w * _NW == n_items
    assert n_outer * _NBUF == per_w
    mesh = plsc.VectorSubcoreMesh(core_axis_name="c", subcore_axis_name="s")

    @functools.partial(
        pl.kernel,
        mesh=mesh,
        out_type=jax.ShapeDtypeStruct(
            (fields * dim // 8, batch // 128, 8, 128), jnp.float32),
        scratch_types=[
            pltpu.VMEM((_NBUF, _CB), jnp.int32),
            pltpu.VMEM((_NBUF, _CB, dim), jnp.float32),
        ]
        + [pltpu.VMEM((dim * _CB // 128, 128), jnp.float32)] * _NBUF
        + [pltpu.SemaphoreType.DMA] * (2 * _NBUF),
        compiler_params=pltpu.CompilerParams(
            use_tc_tiling_on_sc=False, needs_layout_passes=False),
    )
    def gather_kernel(idx_hbm, table_hbm, out_hbm, idx_v, rows_v, *rest):
        outt = rest[:_NBUF]
        gsems = rest[_NBUF:2 * _NBUF]
        osems = rest[2 * _NBUF:]
        wid = lax.axis_index("s") * _NC + lax.axis_index("c")
        base = wid * per_w
        iota16 = lax.iota(jnp.int32, 16)

        def start_gather(k, b):
            pltpu.sync_copy(idx_hbm.at[pl.ds((base + k) * _CB, _CB)],
                            idx_v.at[b])
            pltpu.async_copy(table_hbm.at[idx_v.at[b]], rows_v.at[b], gsems[b])

        for b in range(_NBUF):
            start_gather(b, b)

        def outer(o, carry):
            for b in range(_NBUF):
                k = o * _NBUF + b
                item = base + k
                f = item // n_chunks
                b0 = (item % n_chunks) * _CB
                pltpu.make_async_copy(
                    table_hbm.at[idx_v.at[b]], rows_v.at[b], gsems[b]).wait()

                def blk(i, c):
                    rows = iota16 + i * 16
                    for j in range(dim):
                        v = plsc.load_gather(
                            rows_v.at[b],
                            [rows, jnp.full((16,), j, jnp.int32)])
                        outt[b][j * (_CB // 128) + i // 8,
                                pl.ds((i % 8) * 16, 16)] = v
                    return c

                lax.fori_loop(0, _CB // 16, blk, 0, unroll=False)

                stores = [
                    pltpu.make_async_copy(
                        outt[b].at[pl.ds(j * (_CB // 128), _CB // 128), :],
                        out_hbm.at[f * (dim // 8) + j // 8,
                                   pl.ds(b0 // 128, _CB // 128), j % 8, :],
                        osems[b])
                    for j in range(dim)
                ]
                for s in stores:
                    s.start()
                for s in stores:
                    s.wait()

                @pl.when(o + 1 < n_outer)
                def _():
                    start_gather(k + _NBUF, b)
            return carry

        lax.fori_loop(0, n_outer, outer, 0, unroll=False)

    return gather_kernel


_RC = 128
_RBUF = 4


def _make_retile(vocab, dim):
    n_full = vocab // _RC
    tail = vocab - n_full * _RC
    max_t = (n_full + _NW - 1) // _NW
    mesh = plsc.VectorSubcoreMesh(core_axis_name="c", subcore_axis_name="s")

    @functools.partial(
        pl.kernel,
        mesh=mesh,
        out_type=jax.ShapeDtypeStruct((vocab * dim // 128, 128),
                                      jnp.float32),
        scratch_types=[pltpu.VMEM((dim, _RC), jnp.float32)] * (2 * _RBUF)
        + [pltpu.SemaphoreType.DMA] * (2 * _RBUF)
        + [pltpu.VMEM((dim, 64), jnp.float32),
           pltpu.VMEM((8, 128), jnp.float32)],
        compiler_params=pltpu.CompilerParams(
            use_tc_tiling_on_sc=True, needs_layout_passes=False),
    )
    def retile_kernel(tt_hbm, out_hbm, *rest):
        ins = rest[:_RBUF]
        outt = rest[_RBUF:2 * _RBUF]
        gsems = rest[2 * _RBUF:3 * _RBUF]
        osems = rest[3 * _RBUF:4 * _RBUF]
        tin, tout = rest[4 * _RBUF], rest[4 * _RBUF + 1]
        wid = lax.axis_index("s") * _NC + lax.axis_index("c")
        iota16 = lax.iota(jnp.int32, dim)

        def chunk_id(t):
            return wid + t * _NW

        def start_load(t, b):
            c = chunk_id(t)

            @pl.when(c < n_full)
            def _():
                pltpu.make_async_copy(
                    tt_hbm.at[:, pl.ds(c * _RC, _RC)], ins[b],
                    gsems[b]).start()

        for b in range(min(_RBUF, max_t)):
            start_load(b, b)

        def transpose(src, dst, width):
            def body(v, carry):
                vals = plsc.load_gather(
                    src, [iota16, jnp.full((dim,), v, jnp.int32)])
                dst[v // 8, pl.ds((v % 8) * dim, dim)] = vals
                return carry

            lax.fori_loop(0, width, body, 0, unroll=False)

        n_rounds = (max_t + _RBUF - 1) // _RBUF

        def outer(o, carry):
            for b in range(_RBUF):
                t = o * _RBUF + b
                c = chunk_id(t)

                @pl.when(c < n_full)
                def _():
                    pltpu.make_async_copy(
                        tt_hbm.at[:, pl.ds(c * _RC, _RC)], ins[b],
                        gsems[b]).wait()

                    @pl.when(t >= _RBUF)
                    def _():
                        pltpu.make_async_copy(
                            outt[b],
                            out_hbm.at[pl.ds(0, _RC * dim // 128), :],
                            osems[b]).wait()

                    transpose(ins[b], outt[b], _RC)
                    pltpu.make_async_copy(
                        outt[b],
                        out_hbm.at[pl.ds(c * (_RC * dim // 128),
                                         _RC * dim // 128), :],
                        osems[b]).start()

                @pl.when(t + _RBUF < max_t)
                def _():
                    start_load(t + _RBUF, b)
            return carry

        lax.fori_loop(0, n_rounds, outer, 0, unroll=False)

        for b in range(_RBUF):
            pltpu.make_async_copy(
                outt[b], out_hbm.at[pl.ds(0, _RC * dim // 128), :],
                osems[b]).wait()

        if tail:
            @pl.when(wid == 0)
            def _():
                pltpu.sync_copy(tt_hbm.at[:, pl.ds(n_full * _RC, tail)], tin)
                transpose(tin, tout, tail)
                pltpu.sync_copy(
                    tout,
                    out_hbm.at[pl.ds(n_full * (_RC * dim // 128),
                                     tail * dim // 128), :])

    return retile_kernel


@jax.jit
def kernel(indices, table):
    batch, fields = indices.shape
    vocab = table.shape[0]
    idx_flat = indices.T.reshape(fields * batch)
    table_rm = _make_retile(vocab, DIM)(table.T).reshape(vocab, DIM)
    out4 = _make_gather(fields, batch, DIM)(idx_flat, table_rm)
    out5 = out4.reshape(fields, DIM // 8, batch // 128, 8, 128)
    return jnp.transpose(out5, (2, 4, 0, 1, 3)).reshape(batch, fields, DIM)
